# baseline (speedup 1.0000x reference)
import functools

import jax
import jax.numpy as jnp
from jax import lax
from jax.experimental import pallas as pl
from jax.experimental.pallas import tpu as pltpu
from jax.experimental.pallas import tpu_sc as plsc

N = 10000
E = 320000
P = 100000
D = 128

NC = 2
NS = 16
NW = NC * NS
L = 16

CHUNK = 320
NPAD = CHUNK * NW
EW = E // NW
G = 80
C = 3200
G2 = 64
SB = 8
NEG = -3.0e38

PW = 3136
PPAD = PW * NW

N_BLK = 1000
E_BLK = 2000


def _mm_body(x_ref, w_ref, o_ref):
    o_ref[...] = jnp.dot(x_ref[...], w_ref[...], preferred_element_type=jnp.float32)


def _mm(x, w, blk):
    n, d = x.shape
    k = w.shape[1]
    return pl.pallas_call(
        _mm_body,
        grid=(n // blk,),
        in_specs=[
            pl.BlockSpec((blk, d), lambda i: (i, 0)),
            pl.BlockSpec((d, k), lambda i: (0, 0)),
        ],
        out_specs=pl.BlockSpec((blk, k), lambda i: (i, 0)),
        out_shape=jax.ShapeDtypeStruct((n, k), jnp.float32),
    )(x, w)


def _lin_body(x_ref, seg_ref, pb_ref, wt_ref, wb_ref, lb_ref, o_ref):
    agg = jnp.maximum(seg_ref[...] + pb_ref[...], 0.0)
    h = jnp.dot(x_ref[...], wt_ref[...], preferred_element_type=jnp.float32)
    h = h + jnp.dot(agg, wb_ref[...], preferred_element_type=jnp.float32)
    o_ref[...] = jnp.maximum(h + lb_ref[...], 0.0)


def _fused_lin(x, seg, pool_b, w_top, w_bot, lin_b):
    n, d = x.shape
    k = w_top.shape[1]
    return pl.pallas_call(
        _lin_body,
        grid=(n // N_BLK,),
        in_specs=[
            pl.BlockSpec((N_BLK, d), lambda i: (i, 0)),
            pl.BlockSpec((N_BLK, d), lambda i: (i, 0)),
            pl.BlockSpec((1, d), lambda i: (0, 0)),
            pl.BlockSpec((d, k), lambda i: (0, 0)),
            pl.BlockSpec((d, k), lambda i: (0, 0)),
            pl.BlockSpec((1, k), lambda i: (0, 0)),
        ],
        out_specs=pl.BlockSpec((N_BLK, k), lambda i: (i, 0)),
        out_shape=jax.ShapeDtypeStruct((n, k), jnp.float32),
    )(x, seg, pool_b.reshape(1, d), w_top, w_bot, lin_b.reshape(1, k))


_MESH = plsc.VectorSubcoreMesh(core_axis_name="c", subcore_axis_name="s")


def _wid():
    return lax.axis_index("s") * NC + lax.axis_index("c")


@functools.partial(
    pl.kernel,
    mesh=_MESH,
    compiler_params=pltpu.CompilerParams(needs_layout_passes=False),
    out_type=jax.ShapeDtypeStruct((E, D), jnp.float32),
    scratch_types=[
        pltpu.VMEM((EW,), jnp.int32),
        pltpu.VMEM((EW,), jnp.float32),
        pltpu.VMEM((2, G, D), jnp.float32),
        pltpu.SemaphoreType.DMA,
        pltpu.SemaphoreType.DMA,
    ],
)
def _ef_kernel(x_hbm, src_hbm, scale_hbm, ef_hbm, src_v, sc_v, rows_v, sem0, sem1):
    base = _wid() * EW
    pltpu.sync_copy(src_hbm.at[pl.ds(base, EW)], src_v)
    pltpu.sync_copy(scale_hbm.at[pl.ds(base, EW)], sc_v)

    nb = EW // G

    def fire(b, buf, sem):
        pltpu.async_copy(x_hbm.at[src_v.at[pl.ds(b * G, G)]], rows_v.at[buf], sem)

    def drain(buf, sem):
        pltpu.make_async_copy(x_hbm.at[src_v.at[pl.ds(0, G)]], rows_v.at[buf], sem).wait()

    def process(b, buf):
        def body(gg, _):
            svec = sc_v[pl.ds(b * G + gg * L, L)]
            for i in range(L):
                s = svec[i]
                for j in range(D // L):
                    sl = pl.ds(j * L, L)
                    rows_v[buf, gg * L + i, sl] = rows_v[buf, gg * L + i, sl] * s
            return 0
        lax.fori_loop(0, G // L, body, 0)
        pltpu.sync_copy(rows_v.at[buf], ef_hbm.at[pl.ds(base + b * G, G)])

    fire(0, 0, sem0)

    def loop(k, _):
        fire(2 * k + 1, 1, sem1)
        drain(0, sem0)
        process(2 * k, 0)

        @pl.when(2 * k + 2 < nb)
        def _():
            fire(2 * k + 2, 0, sem0)

        drain(1, sem1)
        process(2 * k + 1, 1)
        return 0

    lax.fori_loop(0, nb // 2, loop, 0)
    drain(0, sem0)
    process(nb - 1, 0)


@functools.partial(
    pl.kernel,
    mesh=_MESH,
    compiler_params=pltpu.CompilerParams(needs_layout_passes=False),
    out_type=jax.ShapeDtypeStruct((NPAD, D), jnp.float32),
    scratch_types=[
        pltpu.VMEM((CHUNK + 1, D), jnp.float32),
        pltpu.VMEM((C,), jnp.int32),
        pltpu.VMEM((C + G,), jnp.int32),
        pltpu.VMEM((C + G,), jnp.int32),
        pltpu.VMEM((2, G, D), jnp.float32),
        pltpu.SemaphoreType.DMA,
        pltpu.SemaphoreType.DMA,
    ],
)
def _segmax_kernel(val_hbm, dst_hbm, seg_hbm, acc, dstst, eidl, dstl, rows, sem0, sem1):
    base = _wid() * CHUNK

    neg = jnp.full((L,), NEG, dtype=jnp.float32)

    def init(i, _):
        acc[i // (D // L), pl.ds((i % (D // L)) * L, L)] = neg
        return 0
    lax.fori_loop(0, (CHUNK + 1) * (D // L), init, 0)

    lanes = lax.iota(jnp.int32, L)
    trash = jnp.full((L,), CHUNK, dtype=jnp.int32)
    zeros = jnp.zeros((L,), dtype=jnp.int32)
    sbstep = jnp.full((L,), SB * L, dtype=jnp.int32)
    offs = [jnp.full((L,), t * L, dtype=jnp.int32) for t in range(SB)]
    npairs = (C // G) // 2 + 1

    def chunk_body(cidx, _):
        pltpu.sync_copy(dst_hbm.at[pl.ds(cidx * C, C)], dstst)
        blo = jnp.full((L,), base, jnp.int32)
        bhi = jnp.full((L,), base + CHUNK, jnp.int32)
        eid0 = jnp.full((L,), cidx * C, jnp.int32) + lanes

        def scan(sb, carry):
            o, eidv = carry
            dvs, ms, cnts = [], [], []
            for t in range(SB):
                dv = dstst[pl.ds(sb * (SB * L) + t * L, L)]
                m = (dv >= blo) & (dv < bhi)
                dvs.append(dv)
                ms.append(m)
                cnts.append(plsc.all_reduce_population_count(m)[0])
            for t in range(SB):
                plsc.store_compressed(eidl.at[pl.ds(o, L)], eidv + offs[t], mask=ms[t])
                plsc.store_compressed(dstl.at[pl.ds(o, L)], dvs[t] - blo, mask=ms[t])
                o = o + cnts[t]
            return (o, eidv + sbstep)
        o, _unused = lax.fori_loop(0, C // (SB * L), scan, (0, eid0))

        def pad(t, _):
            eidl[pl.ds(o + t * L, L)] = zeros
            dstl[pl.ds(o + t * L, L)] = trash
            return 0
        lax.fori_loop(0, G // L, pad, 0)
        nb = (o + G - 1) // G

        def fire(q, buf, sem):
            pltpu.async_copy(val_hbm.at[eidl.at[pl.ds(q * G, G)]], rows.at[buf], sem)

        def drain(buf, sem):
            pltpu.make_async_copy(
                val_hbm.at[eidl.at[pl.ds(0, G)]], rows.at[buf], sem).wait()

        def upd(q, buf):
            def gbody(gg, _):
                rvec = dstl[pl.ds(q * G + gg * L, L)]
                for i in range(L):
                    r = rvec[i]
                    for j in range(D // L):
                        sl = pl.ds(j * L, L)
                        acc[r, sl] = jnp.maximum(acc[r, sl], rows[buf, gg * L + i, sl])
                return 0
            lax.fori_loop(0, G // L, gbody, 0)

        @pl.when(nb > 0)
        def _():
            fire(0, 0, sem0)

        def pair(p, _):
            @pl.when(2 * p + 1 < nb)
            def _():
                fire(2 * p + 1, 1, sem1)

            @pl.when(2 * p < nb)
            def _():
                drain(0, sem0)
                upd(2 * p, 0)

            @pl.when(2 * p + 2 < nb)
            def _():
                fire(2 * p + 2, 0, sem0)

            @pl.when(2 * p + 1 < nb)
            def _():
                drain(1, sem1)
                upd(2 * p + 1, 1)
            return 0
        lax.fori_loop(0, npairs, pair, 0)
        return 0

    lax.fori_loop(0, E // C, chunk_body, 0)
    pltpu.sync_copy(acc.at[pl.ds(0, CHUNK)], seg_hbm.at[pl.ds(base, CHUNK)])


@functools.partial(
    pl.kernel,
    mesh=_MESH,
    compiler_params=pltpu.CompilerParams(needs_layout_passes=False),
    out_type=(
        jax.ShapeDtypeStruct((PPAD,), jnp.float32),
        jax.ShapeDtypeStruct((PPAD,), jnp.float32),
    ),
    scratch_types=[
        pltpu.VMEM((G2, D), jnp.float32),
        pltpu.VMEM((G2, D), jnp.float32),
        pltpu.VMEM((D, 2), jnp.float32),
        pltpu.VMEM((PW,), jnp.int32),
        pltpu.VMEM((PW,), jnp.int32),
        pltpu.VMEM((PW,), jnp.float32),
        pltpu.VMEM((PW,), jnp.float32),
        pltpu.VMEM((2 * L,), jnp.float32),
        pltpu.SemaphoreType.DMA,
        pltpu.SemaphoreType.DMA,
    ],
)
def _edge_head_kernel(h2_hbm, pe0_hbm, pe1_hbm, w_hbm, bias_hbm, ew_hbm, ep_hbm,
                      rows0, rows1, wv, p0, p1, ewv, epv, bv, sem0, sem1):
    base = _wid() * PW
    pltpu.sync_copy(pe0_hbm.at[pl.ds(base, PW)], p0)
    pltpu.sync_copy(pe1_hbm.at[pl.ds(base, PW)], p1)
    pltpu.sync_copy(w_hbm, wv)
    pltpu.sync_copy(bias_hbm, bv)
    bw = bv[pl.ds(0, L)]
    bp = bv[pl.ds(L, L)]

    lanes = lax.iota(jnp.int32, L)
    ivs = [lanes + jnp.full((L,), g2 * L, jnp.int32) for g2 in range(G2 // L)]
    zero16 = jnp.zeros((L,), jnp.int32)
    one16i = jnp.ones((L,), jnp.int32)
    zf = jnp.zeros((L,), jnp.float32)
    rbias = jnp.full((L,), 0x7FFF, jnp.int32)
    rmask = jnp.full((L,), -65536, jnp.int32)

    def rnd_bf16(v):
        iw = plsc.bitcast(v, jnp.int32)
        iw = iw + rbias + ((iw >> 16) & one16i)
        return plsc.bitcast(iw & rmask, jnp.float32)

    def batch(q, _):
        pltpu.async_copy(h2_hbm.at[p0.at[pl.ds(q * G2, G2)]], rows0, sem0)
        pltpu.async_copy(h2_hbm.at[p1.at[pl.ds(q * G2, G2)]], rows1, sem1)
        pltpu.make_async_copy(h2_hbm.at[p0.at[pl.ds(0, G2)]], rows0, sem0).wait()
        pltpu.make_async_copy(h2_hbm.at[p1.at[pl.ds(0, G2)]], rows1, sem1).wait()
        for g2 in range(G2 // L):
            iv = ivs[g2]

            def fbody(f, carry):
                accw, accp, fv = carry
                a0 = plsc.load_gather(rows0, [iv, fv])
                a1 = plsc.load_gather(rows1, [iv, fv])
                ww = plsc.load_gather(wv, [fv, zero16])
                wp = plsc.load_gather(wv, [fv, one16i])
                eb = rnd_bf16(a0 + a1)
                return (accw + eb * ww, accp + eb * wp, fv + one16i)
            accw, accp, _unused = lax.fori_loop(
                0, D, fbody, (zf, zf, zero16))
            sl = pl.ds(q * G2 + g2 * L, L)
            ewv[sl] = jnp.maximum(accw + bw, 0.0)
            epv[sl] = accp + bp
        return 0
    lax.fori_loop(0, PW // G2, batch, 0)

    pltpu.sync_copy(ewv, ew_hbm.at[pl.ds(base, PW)])
    pltpu.sync_copy(epv, ep_hbm.at[pl.ds(base, PW)])


def _conv(x, src, dst, scale, pool_w, pool_b, lin_w, lin_b):
    ef = _ef_kernel(x, src, scale)
    pooledraw = _mm(ef, pool_w, E_BLK)
    seg = _segmax_kernel(pooledraw, dst)[:N]
    d = x.shape[1]
    return _fused_lin(x, seg, pool_b, lin_w[:d], lin_w[d:], lin_b)


def kernel(x, prediction_edges, message_edges, message_edgewt,
           pool1_w, pool1_b, coef1, lin1_w, lin1_b,
           pool2_w, pool2_b, coef2, lin2_w, lin2_b,
           ewp_w, ewp_b, ep_w, ep_b):
    src, dst = message_edges[0], message_edges[1]

    scale1 = 1.0 + coef1 * message_edgewt
    h1 = _conv(x, src, dst, scale1, pool1_w, pool1_b, lin1_w, lin1_b)

    scale2 = 1.0 + coef2 * message_edgewt
    h2 = _conv(h1, src, dst, scale2, pool2_w, pool2_b, lin2_w, lin2_b)

    wb = jnp.concatenate([ewp_w, ep_w], axis=1).astype(jnp.bfloat16).astype(jnp.float32)
    pe0 = jnp.zeros((PPAD,), jnp.int32).at[:P].set(prediction_edges[0])
    pe1 = jnp.zeros((PPAD,), jnp.int32).at[:P].set(prediction_edges[1])
    bias = jnp.concatenate([
        jnp.full((L,), ewp_b[0], jnp.float32),
        jnp.full((L,), ep_b[0], jnp.float32),
    ])
    ew, ep = _edge_head_kernel(h2, pe0, pe1, wb, bias)
    return (ew[:P].reshape(P, 1), ep[:P].reshape(P, 1))


# speedup vs baseline: 1.0012x; 1.0012x over previous
import functools

import jax
import jax.numpy as jnp
from jax import lax
from jax.experimental import pallas as pl
from jax.experimental.pallas import tpu as pltpu
from jax.experimental.pallas import tpu_sc as plsc

N = 10000
E = 320000
P = 100000
D = 128

NC = 2
NS = 16
NW = NC * NS
L = 16

CHUNK = 320
NPAD = CHUNK * NW
EW = E // NW
G = 80
C = 3200
G2 = 64
SB = 8
NEG = -3.0e38

PW = 3136
PPAD = PW * NW

N_BLK = 1000
E_BLK = 2000


def _mm_body(x_ref, w_ref, o_ref):
    o_ref[...] = jnp.dot(x_ref[...], w_ref[...], preferred_element_type=jnp.float32)


def _mm(x, w, blk):
    n, d = x.shape
    k = w.shape[1]
    return pl.pallas_call(
        _mm_body,
        grid=(n // blk,),
        in_specs=[
            pl.BlockSpec((blk, d), lambda i: (i, 0)),
            pl.BlockSpec((d, k), lambda i: (0, 0)),
        ],
        out_specs=pl.BlockSpec((blk, k), lambda i: (i, 0)),
        out_shape=jax.ShapeDtypeStruct((n, k), jnp.float32),
    )(x, w)


def _lin_body(x_ref, seg_ref, pb_ref, wt_ref, wb_ref, lb_ref, o_ref):
    agg = jnp.maximum(seg_ref[...] + pb_ref[...], 0.0)
    h = jnp.dot(x_ref[...], wt_ref[...], preferred_element_type=jnp.float32)
    h = h + jnp.dot(agg, wb_ref[...], preferred_element_type=jnp.float32)
    o_ref[...] = jnp.maximum(h + lb_ref[...], 0.0)


def _fused_lin(x, seg, pool_b, w_top, w_bot, lin_b):
    n, d = x.shape
    k = w_top.shape[1]
    return pl.pallas_call(
        _lin_body,
        grid=(n // N_BLK,),
        in_specs=[
            pl.BlockSpec((N_BLK, d), lambda i: (i, 0)),
            pl.BlockSpec((N_BLK, d), lambda i: (i, 0)),
            pl.BlockSpec((1, d), lambda i: (0, 0)),
            pl.BlockSpec((d, k), lambda i: (0, 0)),
            pl.BlockSpec((d, k), lambda i: (0, 0)),
            pl.BlockSpec((1, k), lambda i: (0, 0)),
        ],
        out_specs=pl.BlockSpec((N_BLK, k), lambda i: (i, 0)),
        out_shape=jax.ShapeDtypeStruct((n, k), jnp.float32),
    )(x, seg, pool_b.reshape(1, d), w_top, w_bot, lin_b.reshape(1, k))


_MESH = plsc.VectorSubcoreMesh(core_axis_name="c", subcore_axis_name="s")


def _wid():
    return lax.axis_index("s") * NC + lax.axis_index("c")


@functools.partial(
    pl.kernel,
    mesh=_MESH,
    compiler_params=pltpu.CompilerParams(needs_layout_passes=False),
    out_type=jax.ShapeDtypeStruct((E, D), jnp.float32),
    scratch_types=[
        pltpu.VMEM((EW,), jnp.int32),
        pltpu.VMEM((EW,), jnp.float32),
        pltpu.VMEM((2, G, D), jnp.float32),
        pltpu.SemaphoreType.DMA,
        pltpu.SemaphoreType.DMA,
    ],
)
def _ef_kernel(x_hbm, src_hbm, scale_hbm, ef_hbm, src_v, sc_v, rows_v, sem0, sem1):
    base = _wid() * EW
    pltpu.sync_copy(src_hbm.at[pl.ds(base, EW)], src_v)
    pltpu.sync_copy(scale_hbm.at[pl.ds(base, EW)], sc_v)

    nb = EW // G

    def fire(b, buf, sem):
        pltpu.async_copy(x_hbm.at[src_v.at[pl.ds(b * G, G)]], rows_v.at[buf], sem)

    def drain(buf, sem):
        pltpu.make_async_copy(x_hbm.at[src_v.at[pl.ds(0, G)]], rows_v.at[buf], sem).wait()

    def process(b, buf):
        def body(gg, _):
            svec = sc_v[pl.ds(b * G + gg * L, L)]
            for i in range(L):
                s = svec[i]
                for j in range(D // L):
                    sl = pl.ds(j * L, L)
                    rows_v[buf, gg * L + i, sl] = rows_v[buf, gg * L + i, sl] * s
            return 0
        lax.fori_loop(0, G // L, body, 0)
        pltpu.sync_copy(rows_v.at[buf], ef_hbm.at[pl.ds(base + b * G, G)])

    fire(0, 0, sem0)

    def loop(k, _):
        fire(2 * k + 1, 1, sem1)
        drain(0, sem0)
        process(2 * k, 0)

        @pl.when(2 * k + 2 < nb)
        def _():
            fire(2 * k + 2, 0, sem0)

        drain(1, sem1)
        process(2 * k + 1, 1)
        return 0

    lax.fori_loop(0, nb // 2, loop, 0)
    drain(0, sem0)
    process(nb - 1, 0)


@functools.partial(
    pl.kernel,
    mesh=_MESH,
    compiler_params=pltpu.CompilerParams(needs_layout_passes=False),
    out_type=jax.ShapeDtypeStruct((NPAD, D), jnp.float32),
    scratch_types=[
        pltpu.VMEM((CHUNK + 1, D), jnp.float32),
        pltpu.VMEM((C,), jnp.int32),
        pltpu.VMEM((C + G,), jnp.int32),
        pltpu.VMEM((C + G,), jnp.int32),
        pltpu.VMEM((2, G, D), jnp.float32),
        pltpu.SemaphoreType.DMA,
        pltpu.SemaphoreType.DMA,
    ],
)
def _segmax_kernel(val_hbm, dst_hbm, seg_hbm, acc, dstst, eidl, dstl, rows, sem0, sem1):
    base = _wid() * CHUNK

    neg = jnp.full((L,), NEG, dtype=jnp.float32)

    def init(i, _):
        acc[i // (D // L), pl.ds((i % (D // L)) * L, L)] = neg
        return 0
    lax.fori_loop(0, (CHUNK + 1) * (D // L), init, 0)

    lanes = lax.iota(jnp.int32, L)
    trash = jnp.full((L,), CHUNK, dtype=jnp.int32)
    zeros = jnp.zeros((L,), dtype=jnp.int32)
    lstep = jnp.full((L,), L, dtype=jnp.int32)
    ones = jnp.ones((L,), dtype=jnp.int32)

    def chunk_body(cidx, _):
        pltpu.sync_copy(dst_hbm.at[pl.ds(cidx * C, C)], dstst)
        blo = jnp.full((L,), base, jnp.int32)
        bhi = jnp.full((L,), base + CHUNK, jnp.int32)
        eid0 = jnp.full((L,), cidx * C, jnp.int32) + lanes

        def scan(g, carry):
            o, eid = carry
            dv = dstst[pl.ds(g * L, L)]
            m = (dv >= blo) & (dv < bhi)
            cnt = plsc.cumsum(jnp.where(m, ones, zeros))[L - 1]
            plsc.store_compressed(eidl.at[pl.ds(o, L)], eid, mask=m)
            plsc.store_compressed(dstl.at[pl.ds(o, L)], dv - blo, mask=m)
            return (o + cnt, eid + lstep)
        o, _unused = lax.fori_loop(0, C // L, scan, (0, eid0))

        def pad(t, _):
            eidl[pl.ds(o + t * L, L)] = zeros
            dstl[pl.ds(o + t * L, L)] = trash
            return 0
        lax.fori_loop(0, G // L, pad, 0)
        nb = (o + G - 1) // G

        def batch(q, _):
            pltpu.async_copy(
                val_hbm.at[eidl.at[pl.ds(q * G, G)]], rows.at[0], sem0).wait()

            def upd(gg, _):
                rvec = dstl[pl.ds(q * G + gg * L, L)]
                for i in range(L):
                    r = rvec[i]
                    for j in range(D // L):
                        sl = pl.ds(j * L, L)
                        acc[r, sl] = jnp.maximum(acc[r, sl], rows[0, gg * L + i, sl])
                return 0
            lax.fori_loop(0, G // L, upd, 0)
            return 0
        lax.fori_loop(0, nb, batch, 0)
        return 0

    lax.fori_loop(0, E // C, chunk_body, 0)
    pltpu.sync_copy(acc.at[pl.ds(0, CHUNK)], seg_hbm.at[pl.ds(base, CHUNK)])


@functools.partial(
    pl.kernel,
    mesh=_MESH,
    compiler_params=pltpu.CompilerParams(needs_layout_passes=False),
    out_type=(
        jax.ShapeDtypeStruct((PPAD,), jnp.float32),
        jax.ShapeDtypeStruct((PPAD,), jnp.float32),
    ),
    scratch_types=[
        pltpu.VMEM((G2, D), jnp.float32),
        pltpu.VMEM((G2, D), jnp.float32),
        pltpu.VMEM((D, 2), jnp.float32),
        pltpu.VMEM((PW,), jnp.int32),
        pltpu.VMEM((PW,), jnp.int32),
        pltpu.VMEM((PW,), jnp.float32),
        pltpu.VMEM((PW,), jnp.float32),
        pltpu.VMEM((2 * L,), jnp.float32),
        pltpu.SemaphoreType.DMA,
        pltpu.SemaphoreType.DMA,
    ],
)
def _edge_head_kernel(h2_hbm, pe0_hbm, pe1_hbm, w_hbm, bias_hbm, ew_hbm, ep_hbm,
                      rows0, rows1, wv, p0, p1, ewv, epv, bv, sem0, sem1):
    base = _wid() * PW
    pltpu.sync_copy(pe0_hbm.at[pl.ds(base, PW)], p0)
    pltpu.sync_copy(pe1_hbm.at[pl.ds(base, PW)], p1)
    pltpu.sync_copy(w_hbm, wv)
    pltpu.sync_copy(bias_hbm, bv)
    bw = bv[pl.ds(0, L)]
    bp = bv[pl.ds(L, L)]

    lanes = lax.iota(jnp.int32, L)
    ivs = [lanes + jnp.full((L,), g2 * L, jnp.int32) for g2 in range(G2 // L)]
    zero16 = jnp.zeros((L,), jnp.int32)
    one16i = jnp.ones((L,), jnp.int32)
    zf = jnp.zeros((L,), jnp.float32)
    rbias = jnp.full((L,), 0x7FFF, jnp.int32)
    rmask = jnp.full((L,), -65536, jnp.int32)

    def rnd_bf16(v):
        iw = plsc.bitcast(v, jnp.int32)
        iw = iw + rbias + ((iw >> 16) & one16i)
        return plsc.bitcast(iw & rmask, jnp.float32)

    def batch(q, _):
        pltpu.async_copy(h2_hbm.at[p0.at[pl.ds(q * G2, G2)]], rows0, sem0)
        pltpu.async_copy(h2_hbm.at[p1.at[pl.ds(q * G2, G2)]], rows1, sem1)
        pltpu.make_async_copy(h2_hbm.at[p0.at[pl.ds(0, G2)]], rows0, sem0).wait()
        pltpu.make_async_copy(h2_hbm.at[p1.at[pl.ds(0, G2)]], rows1, sem1).wait()
        for g2 in range(G2 // L):
            iv = ivs[g2]

            def fbody(f, carry):
                accw, accp, fv = carry
                a0 = plsc.load_gather(rows0, [iv, fv])
                a1 = plsc.load_gather(rows1, [iv, fv])
                ww = plsc.load_gather(wv, [fv, zero16])
                wp = plsc.load_gather(wv, [fv, one16i])
                eb = rnd_bf16(a0 + a1)
                return (accw + eb * ww, accp + eb * wp, fv + one16i)
            accw, accp, _unused = lax.fori_loop(
                0, D, fbody, (zf, zf, zero16))
            sl = pl.ds(q * G2 + g2 * L, L)
            ewv[sl] = jnp.maximum(accw + bw, 0.0)
            epv[sl] = accp + bp
        return 0
    lax.fori_loop(0, PW // G2, batch, 0)

    pltpu.sync_copy(ewv, ew_hbm.at[pl.ds(base, PW)])
    pltpu.sync_copy(epv, ep_hbm.at[pl.ds(base, PW)])


def _conv(x, src, dst, scale, pool_w, pool_b, lin_w, lin_b):
    ef = _ef_kernel(x, src, scale)
    pooledraw = _mm(ef, pool_w, E_BLK)
    seg = _segmax_kernel(pooledraw, dst)[:N]
    d = x.shape[1]
    return _fused_lin(x, seg, pool_b, lin_w[:d], lin_w[d:], lin_b)


def kernel(x, prediction_edges, message_edges, message_edgewt,
           pool1_w, pool1_b, coef1, lin1_w, lin1_b,
           pool2_w, pool2_b, coef2, lin2_w, lin2_b,
           ewp_w, ewp_b, ep_w, ep_b):
    src, dst = message_edges[0], message_edges[1]

    scale1 = 1.0 + coef1 * message_edgewt
    h1 = _conv(x, src, dst, scale1, pool1_w, pool1_b, lin1_w, lin1_b)

    scale2 = 1.0 + coef2 * message_edgewt
    h2 = _conv(h1, src, dst, scale2, pool2_w, pool2_b, lin2_w, lin2_b)

    wb = jnp.concatenate([ewp_w, ep_w], axis=1).astype(jnp.bfloat16).astype(jnp.float32)
    pe0 = jnp.zeros((PPAD,), jnp.int32).at[:P].set(prediction_edges[0])
    pe1 = jnp.zeros((PPAD,), jnp.int32).at[:P].set(prediction_edges[1])
    bias = jnp.concatenate([
        jnp.full((L,), ewp_b[0], jnp.float32),
        jnp.full((L,), ep_b[0], jnp.float32),
    ])
    ew, ep = _edge_head_kernel(h2, pe0, pe1, wb, bias)
    return (ew[:P].reshape(P, 1), ep[:P].reshape(P, 1))
